# baseline (speedup 1.0000x reference)
import jax
import jax.numpy as jnp
from jax import lax
from jax.experimental import pallas as pl
from jax.experimental.pallas import tpu as pltpu
from jax.experimental.pallas import tpu_sc as plsc

EPS_ = 1e-5
NC = 2
NS = 16


def kernel(high_dim_features, low_dim_features, edge_index,
           high_dim_cov_matrix_flatten,
           bn_low_g, bn_low_b, mlp_low_W, mlp_low_b, mlp_low_bn_g,
           mlp_low_bn_b, bn_high_g, bn_high_b, mlp_high_W, mlp_high_b,
           mlp_high_bn_g, mlp_high_bn_b, conv1_W, conv1_b, cls_W, cls_b):
    f32 = jnp.float32
    xc = high_dim_cov_matrix_flatten
    xl = low_dim_features
    N, HD = xc.shape
    LD = xl.shape[1]
    EMB = mlp_low_W.shape[1]
    HID = conv1_W.shape[1]
    OUT = cls_W.shape[1]
    E = edge_index.shape[1]

    CH = 128
    NW = NC * NS
    CHW = CH * NW
    NF = -(-E // CHW)
    NF = ((NF + 7) // 8) * 8
    E_PAD = NF * CHW
    NCH = E_PAD // CH
    NPH = 2
    NF2 = NF // NPH
    NP8 = NS * 8
    N_PAD = ((N + 8 + NP8 - 1) // NP8) * NP8
    RPT = N_PAD // NS

    RB = 1000
    NB = N // RB

    r = lambda v: v.reshape(1, -1)
    mesh = plsc.VectorSubcoreMesh(core_axis_name="c", subcore_axis_name="s",
                                  num_cores=NC, num_subcores=NS)

    npad_e = E_PAD - E
    pad_i = jnp.arange(npad_e, dtype=jnp.int32)
    src2 = jnp.concatenate(
        [edge_index[0], pad_i % N]).reshape(NCH, CH)
    dst2 = jnp.concatenate(
        [edge_index[1], N + (pad_i % 96)]).reshape(NCH, CH)

    RPT1 = ((N + NS * 128 - 1) // (NS * 128)) * 128
    N_PAD1 = RPT1 * NS

    def _deg_body(dst_hbm, ones_hbm, zeros_hbm, out_hbm, dstv, onesv, acc,
                  ssem):
        c = lax.axis_index("c")
        s = lax.axis_index("s")
        w = c * NS + s
        pltpu.sync_copy(zeros_hbm.at[pl.ds(s * RPT1, RPT1)],
                        acc.at[pl.ds(s * RPT1, RPT1)])
        pltpu.sync_copy(ones_hbm, onesv)
        pltpu.sync_copy(dst_hbm.at[pl.ds(w * NF, NF)], dstv)
        plsc.subcore_barrier()

        def step(j, carry):
            pltpu.async_copy(onesv, acc.at[dstv.at[j]], ssem, add=True)

            @pl.when(j >= 4)
            def _():
                pltpu.make_async_copy(onesv, acc.at[dstv.at[0]], ssem).wait()

            return carry

        lax.fori_loop(0, NF, step, 0)

        def drain(j, carry):
            pltpu.make_async_copy(onesv, acc.at[dstv.at[0]], ssem).wait()
            return carry

        lax.fori_loop(0, 4, drain, 0)
        plsc.subcore_barrier()
        pltpu.sync_copy(acc.at[pl.ds(s * RPT1, RPT1)],
                        out_hbm.at[pl.ds(w * RPT1, RPT1)])

    deg_fn = pl.kernel(
        _deg_body,
        out_type=jax.ShapeDtypeStruct((NW * RPT1,), f32),
        mesh=mesh,
        scratch_types=[
            pltpu.VMEM((NF, CH), jnp.int32),
            pltpu.VMEM((CH,), f32),
            pltpu.VMEM_SHARED((N_PAD1,), f32),
            pltpu.SemaphoreType.DMA,
        ],
    )
    degp = deg_fn(dst2, jnp.ones((CH,), f32),
                  jnp.zeros((N_PAD1,), f32)).reshape(NC, N_PAD1)
    p0 = degp[0, :N].reshape(N, 1)
    p1 = degp[1, :N].reshape(N, 1)

    def _stats_body(xc_ref, xl_ref, sc_ref, sqc_ref, sl_ref, sql_ref):
        @pl.when(pl.program_id(0) == 0)
        def _():
            sc_ref[...] = jnp.zeros_like(sc_ref)
            sqc_ref[...] = jnp.zeros_like(sqc_ref)
            sl_ref[...] = jnp.zeros_like(sl_ref)
            sql_ref[...] = jnp.zeros_like(sql_ref)

        a = xc_ref[...]
        b = xl_ref[...]
        sc_ref[...] += jnp.sum(a, axis=0, keepdims=True)
        sqc_ref[...] += jnp.sum(a * a, axis=0, keepdims=True)
        sl_ref[...] += jnp.sum(b, axis=0, keepdims=True)
        sql_ref[...] += jnp.sum(b * b, axis=0, keepdims=True)

    sc_, sqc_, sl_, sql_ = pl.pallas_call(
        _stats_body,
        grid=(NB,),
        in_specs=[
            pl.BlockSpec((RB, HD), lambda i: (i, 0)),
            pl.BlockSpec((RB, LD), lambda i: (i, 0)),
        ],
        out_specs=[
            pl.BlockSpec((1, HD), lambda i: (0, 0)),
            pl.BlockSpec((1, HD), lambda i: (0, 0)),
            pl.BlockSpec((1, LD), lambda i: (0, 0)),
            pl.BlockSpec((1, LD), lambda i: (0, 0)),
        ],
        out_shape=[
            jax.ShapeDtypeStruct((1, HD), f32),
            jax.ShapeDtypeStruct((1, HD), f32),
            jax.ShapeDtypeStruct((1, LD), f32),
            jax.ShapeDtypeStruct((1, LD), f32),
        ],
    )(xc, xl)

    ninv = 1.0 / N

    def _embed_body(xc_ref, xl_ref, sc, sqc, sl, sql, g1h, b1h, Wh, bmh,
                    g1l, b1l, Wl, bml, y_ref, ys_ref, ysq_ref):
        mh = sc[...] * ninv
        vh = sqc[...] * ninv - mh * mh
        ah = g1h[...] * lax.rsqrt(vh + EPS_)
        chn = b1h[...] - mh * ah
        yh = jnp.maximum(
            jnp.dot(xc_ref[...] * ah + chn, Wh[...],
                    preferred_element_type=f32) + bmh[...], 0.0)
        ml = sl[...] * ninv
        vl = sql[...] * ninv - ml * ml
        al = g1l[...] * lax.rsqrt(vl + EPS_)
        cln = b1l[...] - ml * al
        yl = jnp.maximum(
            jnp.dot(xl_ref[...] * al + cln, Wl[...],
                    preferred_element_type=f32) + bml[...], 0.0)
        y = jnp.concatenate([yh, yl], axis=1)
        y_ref[...] = y

        @pl.when(pl.program_id(0) == 0)
        def _():
            ys_ref[...] = jnp.zeros_like(ys_ref)
            ysq_ref[...] = jnp.zeros_like(ysq_ref)

        ys_ref[...] += jnp.sum(y, axis=0, keepdims=True)
        ysq_ref[...] += jnp.sum(y * y, axis=0, keepdims=True)

    vec = lambda K: pl.BlockSpec((1, K), lambda i: (0, 0))
    full = lambda a, b: pl.BlockSpec((a, b), lambda i: (0, 0))
    y_, ys_, ysq_ = pl.pallas_call(
        _embed_body,
        grid=(NB,),
        in_specs=[
            pl.BlockSpec((RB, HD), lambda i: (i, 0)),
            pl.BlockSpec((RB, LD), lambda i: (i, 0)),
            vec(HD), vec(HD), vec(LD), vec(LD),
            vec(HD), vec(HD), full(HD, EMB), vec(EMB),
            vec(LD), vec(LD), full(LD, EMB), vec(EMB),
        ],
        out_specs=[
            pl.BlockSpec((RB, 2 * EMB), lambda i: (i, 0)),
            vec(2 * EMB), vec(2 * EMB),
        ],
        out_shape=[
            jax.ShapeDtypeStruct((N, 2 * EMB), f32),
            jax.ShapeDtypeStruct((1, 2 * EMB), f32),
            jax.ShapeDtypeStruct((1, 2 * EMB), f32),
        ],
    )(xc, xl, sc_, sqc_, sl_, sql_,
      r(bn_high_g), r(bn_high_b), mlp_high_W, r(mlp_high_b),
      r(bn_low_g), r(bn_low_b), mlp_low_W, r(mlp_low_b))

    g2 = r(jnp.concatenate([mlp_high_bn_g, mlp_low_bn_g]))
    b2 = r(jnp.concatenate([mlp_high_bn_b, mlp_low_bn_b]))

    def _hprime_body(y_ref, ys, ysq, g2r, b2r, W, p0r, p1r, hp_ref):
        my = ys[...] * ninv
        vy = ysq[...] * ninv - my * my
        a2 = g2r[...] * lax.rsqrt(vy + EPS_)
        c2 = b2r[...] - my * a2
        t = y_ref[...] * a2 + c2
        h = jnp.dot(t, W[...], preferred_element_type=f32)
        deg = 1.0 + p0r[...] + p1r[...]
        hp_ref[...] = lax.rsqrt(deg) * h

    hp = pl.pallas_call(
        _hprime_body,
        grid=(NB,),
        in_specs=[
            pl.BlockSpec((RB, 2 * EMB), lambda i: (i, 0)),
            vec(2 * EMB), vec(2 * EMB), vec(2 * EMB), vec(2 * EMB),
            full(2 * EMB, HID),
            pl.BlockSpec((RB, 1), lambda i: (i, 0)),
            pl.BlockSpec((RB, 1), lambda i: (i, 0)),
        ],
        out_specs=pl.BlockSpec((RB, HID), lambda i: (i, 0)),
        out_shape=jax.ShapeDtypeStruct((N, HID), f32),
    )(y_, ys_, ysq_, g2, b2, conv1_W, p0, p1)

    def _msg_body(hp_hbm, src_hbm, dst_hbm, zeros_hbm, out_hbm,
                  srcv, dstv, rows, acc, gsem0, gsem1, ssem0, ssem1):
        c = lax.axis_index("c")
        s = lax.axis_index("s")
        w = c * NS + s
        pltpu.sync_copy(zeros_hbm.at[pl.ds(s * RPT, RPT)],
                        acc.at[pl.ds(s * RPT, RPT)])
        plsc.subcore_barrier()

        for p in range(NPH):
            base = w * NF + p * NF2
            pltpu.sync_copy(src_hbm.at[pl.ds(base, NF2)], srcv)
            pltpu.sync_copy(dst_hbm.at[pl.ds(base, NF2)], dstv)
            pltpu.async_copy(hp_hbm.at[srcv.at[0]], rows.at[0], gsem0)

            def step(j2, carry):
                j = 2 * j2
                pltpu.make_async_copy(hp_hbm.at[srcv.at[j]], rows.at[0],
                                      gsem0).wait()
                pltpu.async_copy(rows.at[0], acc.at[dstv.at[j]], ssem0,
                                 add=True)

                @pl.when(j2 > 0)
                def _():
                    pltpu.make_async_copy(rows.at[1], acc.at[dstv.at[0]],
                                          ssem1).wait()

                pltpu.async_copy(hp_hbm.at[srcv.at[j + 1]], rows.at[1], gsem1)
                pltpu.make_async_copy(hp_hbm.at[srcv.at[j + 1]], rows.at[1],
                                      gsem1).wait()
                pltpu.async_copy(rows.at[1], acc.at[dstv.at[j + 1]], ssem1,
                                 add=True)

                @pl.when(j2 + 1 < NF2 // 2)
                def _():
                    pltpu.make_async_copy(rows.at[0], acc.at[dstv.at[0]],
                                          ssem0).wait()
                    pltpu.async_copy(hp_hbm.at[srcv.at[j + 2]], rows.at[0],
                                     gsem0)

                return carry

            lax.fori_loop(0, NF2 // 2, step, 0)
            pltpu.make_async_copy(rows.at[0], acc.at[dstv.at[0]], ssem0).wait()
            pltpu.make_async_copy(rows.at[1], acc.at[dstv.at[0]], ssem1).wait()

        plsc.subcore_barrier()
        pltpu.sync_copy(acc.at[pl.ds(s * RPT, RPT)],
                        out_hbm.at[c].at[pl.ds(s * RPT, RPT)])

    msg_fn = pl.kernel(
        _msg_body,
        out_type=jax.ShapeDtypeStruct((NC, N_PAD, HID), f32),
        mesh=mesh,
        scratch_types=[
            pltpu.VMEM((NF2, CH), jnp.int32),
            pltpu.VMEM((NF2, CH), jnp.int32),
            pltpu.VMEM((2, CH, HID), f32),
            pltpu.VMEM_SHARED((N_PAD, HID), f32),
            pltpu.SemaphoreType.DMA,
            pltpu.SemaphoreType.DMA,
            pltpu.SemaphoreType.DMA,
            pltpu.SemaphoreType.DMA,
        ],
    )
    P = msg_fn(hp, src2, dst2, jnp.zeros((N_PAD, HID), f32))

    def _final_body(P0r, P1r, hpr, p0r, p1r, cb, cW, cbb, o_ref):
        deg = 1.0 + p0r[...] + p1r[...]
        dis = lax.rsqrt(deg)
        x1 = dis * (P0r[...] + P1r[...] + hpr[...]) + cb[...]
        t = jnp.tanh(x1)
        z = jnp.dot(t, cW[...], preferred_element_type=f32) + cbb[...]
        m = jnp.max(z, axis=1, keepdims=True)
        e = z - m
        o_ref[...] = e - jnp.log(jnp.sum(jnp.exp(e), axis=1, keepdims=True))

    out = pl.pallas_call(
        _final_body,
        grid=(NB,),
        in_specs=[
            pl.BlockSpec((RB, HID), lambda i: (i, 0)),
            pl.BlockSpec((RB, HID), lambda i: (i, 0)),
            pl.BlockSpec((RB, HID), lambda i: (i, 0)),
            pl.BlockSpec((RB, 1), lambda i: (i, 0)),
            pl.BlockSpec((RB, 1), lambda i: (i, 0)),
            vec(HID), full(HID, OUT), vec(OUT),
        ],
        out_specs=pl.BlockSpec((RB, OUT), lambda i: (i, 0)),
        out_shape=jax.ShapeDtypeStruct((N, OUT), f32),
    )(P[0, :N], P[1, :N], hp, p0, p1, r(conv1_b), cls_W, r(cls_b))

    return out


# speedup vs baseline: 1.1262x; 1.1262x over previous
import jax
import jax.numpy as jnp
from jax import lax
from jax.experimental import pallas as pl
from jax.experimental.pallas import tpu as pltpu
from jax.experimental.pallas import tpu_sc as plsc

EPS_ = 1e-5
NC = 2
NS = 16


def kernel(high_dim_features, low_dim_features, edge_index,
           high_dim_cov_matrix_flatten,
           bn_low_g, bn_low_b, mlp_low_W, mlp_low_b, mlp_low_bn_g,
           mlp_low_bn_b, bn_high_g, bn_high_b, mlp_high_W, mlp_high_b,
           mlp_high_bn_g, mlp_high_bn_b, conv1_W, conv1_b, cls_W, cls_b):
    f32 = jnp.float32
    xc = high_dim_cov_matrix_flatten
    xl = low_dim_features
    N, HD = xc.shape
    LD = xl.shape[1]
    EMB = mlp_low_W.shape[1]
    HID = conv1_W.shape[1]
    OUT = cls_W.shape[1]
    E = edge_index.shape[1]

    CH = 128
    NW = NC * NS
    CHW = CH * NW
    NF = -(-E // CHW)
    NF = ((NF + 7) // 8) * 8
    E_PAD = NF * CHW
    NCH = E_PAD // CH
    NPH = 2
    NF2 = NF // NPH
    NP8 = NS * 8
    N_PAD = ((N + 8 + NP8 - 1) // NP8) * NP8
    RPT = N_PAD // NS

    RB = 1000
    NB = N // RB

    r = lambda v: v.reshape(1, -1)
    mesh = plsc.VectorSubcoreMesh(core_axis_name="c", subcore_axis_name="s",
                                  num_cores=NC, num_subcores=NS)

    npad_e = E_PAD - E
    pad_i = jnp.arange(npad_e, dtype=jnp.int32)
    src2 = jnp.concatenate(
        [edge_index[0], pad_i % N]).reshape(NCH, CH)
    dst2 = jnp.concatenate(
        [edge_index[1], N + (pad_i % 96)]).reshape(NCH, CH)

    RPT1 = ((N + NS * 128 - 1) // (NS * 128)) * 128
    N_PAD1 = RPT1 * NS

    def _deg_body(dst_hbm, ones_hbm, zeros_hbm, out_hbm, dstv, onesv, acc,
                  ssem):
        c = lax.axis_index("c")
        s = lax.axis_index("s")
        w = c * NS + s
        pltpu.sync_copy(zeros_hbm.at[pl.ds(s * RPT1, RPT1)],
                        acc.at[pl.ds(s * RPT1, RPT1)])
        pltpu.sync_copy(ones_hbm, onesv)
        pltpu.sync_copy(dst_hbm.at[pl.ds(w * NF, NF)], dstv)
        plsc.subcore_barrier()

        def step(j, carry):
            pltpu.async_copy(onesv, acc.at[dstv.at[j]], ssem, add=True)

            @pl.when(j >= 4)
            def _():
                pltpu.make_async_copy(onesv, acc.at[dstv.at[0]], ssem).wait()

            return carry

        lax.fori_loop(0, NF, step, 0)

        def drain(j, carry):
            pltpu.make_async_copy(onesv, acc.at[dstv.at[0]], ssem).wait()
            return carry

        lax.fori_loop(0, 4, drain, 0)
        plsc.subcore_barrier()
        pltpu.sync_copy(acc.at[pl.ds(s * RPT1, RPT1)],
                        out_hbm.at[pl.ds(w * RPT1, RPT1)])

    deg_fn = pl.kernel(
        _deg_body,
        out_type=jax.ShapeDtypeStruct((NW * RPT1,), f32),
        mesh=mesh,
        scratch_types=[
            pltpu.VMEM((NF, CH), jnp.int32),
            pltpu.VMEM((CH,), f32),
            pltpu.VMEM_SHARED((N_PAD1,), f32),
            pltpu.SemaphoreType.DMA,
        ],
    )
    degp = deg_fn(dst2, jnp.ones((CH,), f32),
                  jnp.zeros((N_PAD1,), f32)).reshape(NC, N_PAD1)
    p0 = degp[0, :N].reshape(N, 1)
    p1 = degp[1, :N].reshape(N, 1)

    def _stats_body(xc_ref, xl_ref, sc_ref, sqc_ref, sl_ref, sql_ref):
        @pl.when(pl.program_id(0) == 0)
        def _():
            sc_ref[...] = jnp.zeros_like(sc_ref)
            sqc_ref[...] = jnp.zeros_like(sqc_ref)
            sl_ref[...] = jnp.zeros_like(sl_ref)
            sql_ref[...] = jnp.zeros_like(sql_ref)

        a = xc_ref[...]
        b = xl_ref[...]
        sc_ref[...] += jnp.sum(a, axis=0, keepdims=True)
        sqc_ref[...] += jnp.sum(a * a, axis=0, keepdims=True)
        sl_ref[...] += jnp.sum(b, axis=0, keepdims=True)
        sql_ref[...] += jnp.sum(b * b, axis=0, keepdims=True)

    sc_, sqc_, sl_, sql_ = pl.pallas_call(
        _stats_body,
        grid=(NB,),
        in_specs=[
            pl.BlockSpec((RB, HD), lambda i: (i, 0)),
            pl.BlockSpec((RB, LD), lambda i: (i, 0)),
        ],
        out_specs=[
            pl.BlockSpec((1, HD), lambda i: (0, 0)),
            pl.BlockSpec((1, HD), lambda i: (0, 0)),
            pl.BlockSpec((1, LD), lambda i: (0, 0)),
            pl.BlockSpec((1, LD), lambda i: (0, 0)),
        ],
        out_shape=[
            jax.ShapeDtypeStruct((1, HD), f32),
            jax.ShapeDtypeStruct((1, HD), f32),
            jax.ShapeDtypeStruct((1, LD), f32),
            jax.ShapeDtypeStruct((1, LD), f32),
        ],
    )(xc, xl)

    ninv = 1.0 / N

    def _embed_body(xc_ref, xl_ref, sc, sqc, sl, sql, g1h, b1h, Wh, bmh,
                    g1l, b1l, Wl, bml, y_ref, ys_ref, ysq_ref):
        mh = sc[...] * ninv
        vh = sqc[...] * ninv - mh * mh
        ah = g1h[...] * lax.rsqrt(vh + EPS_)
        chn = b1h[...] - mh * ah
        yh = jnp.maximum(
            jnp.dot(xc_ref[...] * ah + chn, Wh[...],
                    preferred_element_type=f32) + bmh[...], 0.0)
        ml = sl[...] * ninv
        vl = sql[...] * ninv - ml * ml
        al = g1l[...] * lax.rsqrt(vl + EPS_)
        cln = b1l[...] - ml * al
        yl = jnp.maximum(
            jnp.dot(xl_ref[...] * al + cln, Wl[...],
                    preferred_element_type=f32) + bml[...], 0.0)
        y = jnp.concatenate([yh, yl], axis=1)
        y_ref[...] = y

        @pl.when(pl.program_id(0) == 0)
        def _():
            ys_ref[...] = jnp.zeros_like(ys_ref)
            ysq_ref[...] = jnp.zeros_like(ysq_ref)

        ys_ref[...] += jnp.sum(y, axis=0, keepdims=True)
        ysq_ref[...] += jnp.sum(y * y, axis=0, keepdims=True)

    vec = lambda K: pl.BlockSpec((1, K), lambda i: (0, 0))
    full = lambda a, b: pl.BlockSpec((a, b), lambda i: (0, 0))
    y_, ys_, ysq_ = pl.pallas_call(
        _embed_body,
        grid=(NB,),
        in_specs=[
            pl.BlockSpec((RB, HD), lambda i: (i, 0)),
            pl.BlockSpec((RB, LD), lambda i: (i, 0)),
            vec(HD), vec(HD), vec(LD), vec(LD),
            vec(HD), vec(HD), full(HD, EMB), vec(EMB),
            vec(LD), vec(LD), full(LD, EMB), vec(EMB),
        ],
        out_specs=[
            pl.BlockSpec((RB, 2 * EMB), lambda i: (i, 0)),
            vec(2 * EMB), vec(2 * EMB),
        ],
        out_shape=[
            jax.ShapeDtypeStruct((N, 2 * EMB), f32),
            jax.ShapeDtypeStruct((1, 2 * EMB), f32),
            jax.ShapeDtypeStruct((1, 2 * EMB), f32),
        ],
    )(xc, xl, sc_, sqc_, sl_, sql_,
      r(bn_high_g), r(bn_high_b), mlp_high_W, r(mlp_high_b),
      r(bn_low_g), r(bn_low_b), mlp_low_W, r(mlp_low_b))

    g2 = r(jnp.concatenate([mlp_high_bn_g, mlp_low_bn_g]))
    b2 = r(jnp.concatenate([mlp_high_bn_b, mlp_low_bn_b]))

    def _hprime_body(y_ref, ys, ysq, g2r, b2r, W, p0r, p1r, hp_ref):
        my = ys[...] * ninv
        vy = ysq[...] * ninv - my * my
        a2 = g2r[...] * lax.rsqrt(vy + EPS_)
        c2 = b2r[...] - my * a2
        t = y_ref[...] * a2 + c2
        h = jnp.dot(t, W[...], preferred_element_type=f32)
        deg = 1.0 + p0r[...] + p1r[...]
        hp_ref[...] = lax.rsqrt(deg) * h

    hp = pl.pallas_call(
        _hprime_body,
        grid=(NB,),
        in_specs=[
            pl.BlockSpec((RB, 2 * EMB), lambda i: (i, 0)),
            vec(2 * EMB), vec(2 * EMB), vec(2 * EMB), vec(2 * EMB),
            full(2 * EMB, HID),
            pl.BlockSpec((RB, 1), lambda i: (i, 0)),
            pl.BlockSpec((RB, 1), lambda i: (i, 0)),
        ],
        out_specs=pl.BlockSpec((RB, HID), lambda i: (i, 0)),
        out_shape=jax.ShapeDtypeStruct((N, HID), f32),
    )(y_, ys_, ysq_, g2, b2, conv1_W, p0, p1)

    def _msg_body(hp_hbm, src_hbm, dst_hbm, zeros_hbm, out_hbm,
                  srcv, dstv, rows, acc, gsem0, gsem1):
        c = lax.axis_index("c")
        s = lax.axis_index("s")
        w = c * NS + s
        pltpu.sync_copy(zeros_hbm.at[pl.ds(s * RPT, RPT)],
                        acc.at[pl.ds(s * RPT, RPT)])
        plsc.subcore_barrier()

        for p in range(NPH):
            base = w * NF + p * NF2
            pltpu.sync_copy(src_hbm.at[pl.ds(base, NF2)], srcv)
            pltpu.sync_copy(dst_hbm.at[pl.ds(base, NF2)], dstv)
            pltpu.async_copy(hp_hbm.at[srcv.at[0]], rows.at[0], gsem0)

            def step(j2, carry):
                j = 2 * j2
                pltpu.async_copy(hp_hbm.at[srcv.at[j + 1]], rows.at[1], gsem1)
                pltpu.make_async_copy(hp_hbm.at[srcv.at[j]], rows.at[0],
                                      gsem0).wait()
                pltpu.sync_copy(rows.at[0], acc.at[dstv.at[j]], add=True)

                @pl.when(j2 + 1 < NF2 // 2)
                def _():
                    pltpu.async_copy(hp_hbm.at[srcv.at[j + 2]], rows.at[0],
                                     gsem0)

                pltpu.make_async_copy(hp_hbm.at[srcv.at[j + 1]], rows.at[1],
                                      gsem1).wait()
                pltpu.sync_copy(rows.at[1], acc.at[dstv.at[j + 1]], add=True)
                return carry

            lax.fori_loop(0, NF2 // 2, step, 0)

        plsc.subcore_barrier()
        pltpu.sync_copy(acc.at[pl.ds(s * RPT, RPT)],
                        out_hbm.at[c].at[pl.ds(s * RPT, RPT)])

    msg_fn = pl.kernel(
        _msg_body,
        out_type=jax.ShapeDtypeStruct((NC, N_PAD, HID), f32),
        mesh=mesh,
        scratch_types=[
            pltpu.VMEM((NF2, CH), jnp.int32),
            pltpu.VMEM((NF2, CH), jnp.int32),
            pltpu.VMEM((2, CH, HID), f32),
            pltpu.VMEM_SHARED((N_PAD, HID), f32),
            pltpu.SemaphoreType.DMA,
            pltpu.SemaphoreType.DMA,
        ],
    )
    P = msg_fn(hp, src2, dst2, jnp.zeros((N_PAD, HID), f32))

    def _final_body(P0r, P1r, hpr, p0r, p1r, cb, cW, cbb, o_ref):
        deg = 1.0 + p0r[...] + p1r[...]
        dis = lax.rsqrt(deg)
        x1 = dis * (P0r[...][0] + P1r[...][0] + hpr[...]) + cb[...]
        t = jnp.tanh(x1)
        z = jnp.dot(t, cW[...], preferred_element_type=f32) + cbb[...]
        m = jnp.max(z, axis=1, keepdims=True)
        e = z - m
        o_ref[...] = e - jnp.log(jnp.sum(jnp.exp(e), axis=1, keepdims=True))

    out = pl.pallas_call(
        _final_body,
        grid=(NB,),
        in_specs=[
            pl.BlockSpec((1, RB, HID), lambda i: (0, i, 0)),
            pl.BlockSpec((1, RB, HID), lambda i: (1, i, 0)),
            pl.BlockSpec((RB, HID), lambda i: (i, 0)),
            pl.BlockSpec((RB, 1), lambda i: (i, 0)),
            pl.BlockSpec((RB, 1), lambda i: (i, 0)),
            vec(HID), full(HID, OUT), vec(OUT),
        ],
        out_specs=pl.BlockSpec((RB, OUT), lambda i: (i, 0)),
        out_shape=jax.ShapeDtypeStruct((N, OUT), f32),
    )(P, P, hp, p0, p1, r(conv1_b), cls_W, r(cls_b))

    return out


# speedup vs baseline: 1.1744x; 1.0427x over previous
import jax
import jax.numpy as jnp
from jax import lax
from jax.experimental import pallas as pl
from jax.experimental.pallas import tpu as pltpu
from jax.experimental.pallas import tpu_sc as plsc

EPS_ = 1e-5
NC = 2
NS = 16


def kernel(high_dim_features, low_dim_features, edge_index,
           high_dim_cov_matrix_flatten,
           bn_low_g, bn_low_b, mlp_low_W, mlp_low_b, mlp_low_bn_g,
           mlp_low_bn_b, bn_high_g, bn_high_b, mlp_high_W, mlp_high_b,
           mlp_high_bn_g, mlp_high_bn_b, conv1_W, conv1_b, cls_W, cls_b):
    f32 = jnp.float32
    xc = high_dim_cov_matrix_flatten
    xl = low_dim_features
    N, HD = xc.shape
    LD = xl.shape[1]
    EMB = mlp_low_W.shape[1]
    HID = conv1_W.shape[1]
    OUT = cls_W.shape[1]
    E = edge_index.shape[1]

    CH = 128
    NW = NC * NS
    CHW = CH * NW
    NF = -(-E // CHW)
    NF = ((NF + 7) // 8) * 8
    E_PAD = NF * CHW
    NCH = E_PAD // CH
    NPH = 2
    NF2 = NF // NPH
    NP8 = NS * 8
    N_PAD = ((N + 8 + NP8 - 1) // NP8) * NP8
    RPT = N_PAD // NS

    RB = 2000
    NB = N // RB

    r = lambda v: v.reshape(1, -1)
    mesh = plsc.VectorSubcoreMesh(core_axis_name="c", subcore_axis_name="s",
                                  num_cores=NC, num_subcores=NS)

    npad_e = E_PAD - E
    pad_i = jnp.arange(npad_e, dtype=jnp.int32)
    src2 = jnp.concatenate(
        [edge_index[0], pad_i % N]).reshape(NCH, CH)
    dst2 = jnp.concatenate(
        [edge_index[1], N + (pad_i % 96)]).reshape(NCH, CH)

    RPT1 = ((N + NS * 128 - 1) // (NS * 128)) * 128
    N_PAD1 = RPT1 * NS

    def _deg_body(dst_hbm, ones_hbm, zeros_hbm, out_hbm, dstv, onesv, acc,
                  ssem):
        c = lax.axis_index("c")
        s = lax.axis_index("s")
        w = c * NS + s
        pltpu.sync_copy(zeros_hbm, acc.at[pl.ds(s * RPT1, RPT1)])
        pltpu.sync_copy(ones_hbm, onesv)
        pltpu.sync_copy(dst_hbm.at[pl.ds(w * NF, NF)], dstv)
        plsc.subcore_barrier()

        def step(j, carry):
            pltpu.async_copy(onesv, acc.at[dstv.at[j]], ssem, add=True)

            @pl.when(j >= 4)
            def _():
                pltpu.make_async_copy(onesv, acc.at[dstv.at[0]], ssem).wait()

            return carry

        lax.fori_loop(0, NF, step, 0)

        def drain(j, carry):
            pltpu.make_async_copy(onesv, acc.at[dstv.at[0]], ssem).wait()
            return carry

        lax.fori_loop(0, 4, drain, 0)
        plsc.subcore_barrier()
        pltpu.sync_copy(acc.at[pl.ds(s * RPT1, RPT1)],
                        out_hbm.at[pl.ds(w * RPT1, RPT1)])

    deg_fn = pl.kernel(
        _deg_body,
        out_type=jax.ShapeDtypeStruct((NW * RPT1,), f32),
        mesh=mesh,
        scratch_types=[
            pltpu.VMEM((NF, CH), jnp.int32),
            pltpu.VMEM((CH,), f32),
            pltpu.VMEM_SHARED((N_PAD1,), f32),
            pltpu.SemaphoreType.DMA,
        ],
    )
    degp = deg_fn(dst2, jnp.ones((CH,), f32),
                  jnp.zeros((RPT1,), f32)).reshape(NC, N_PAD1)
    p0 = degp[0, :N].reshape(N, 1)
    p1 = degp[1, :N].reshape(N, 1)

    def _stats_body(xc_ref, xl_ref, sc_ref, sqc_ref, sl_ref, sql_ref):
        @pl.when(pl.program_id(0) == 0)
        def _():
            sc_ref[...] = jnp.zeros_like(sc_ref)
            sqc_ref[...] = jnp.zeros_like(sqc_ref)
            sl_ref[...] = jnp.zeros_like(sl_ref)
            sql_ref[...] = jnp.zeros_like(sql_ref)

        a = xc_ref[...]
        b = xl_ref[...]
        sc_ref[...] += jnp.sum(a, axis=0, keepdims=True)
        sqc_ref[...] += jnp.sum(a * a, axis=0, keepdims=True)
        sl_ref[...] += jnp.sum(b, axis=0, keepdims=True)
        sql_ref[...] += jnp.sum(b * b, axis=0, keepdims=True)

    sc_, sqc_, sl_, sql_ = pl.pallas_call(
        _stats_body,
        grid=(NB,),
        in_specs=[
            pl.BlockSpec((RB, HD), lambda i: (i, 0)),
            pl.BlockSpec((RB, LD), lambda i: (i, 0)),
        ],
        out_specs=[
            pl.BlockSpec((1, HD), lambda i: (0, 0)),
            pl.BlockSpec((1, HD), lambda i: (0, 0)),
            pl.BlockSpec((1, LD), lambda i: (0, 0)),
            pl.BlockSpec((1, LD), lambda i: (0, 0)),
        ],
        out_shape=[
            jax.ShapeDtypeStruct((1, HD), f32),
            jax.ShapeDtypeStruct((1, HD), f32),
            jax.ShapeDtypeStruct((1, LD), f32),
            jax.ShapeDtypeStruct((1, LD), f32),
        ],
    )(xc, xl)

    ninv = 1.0 / N

    def _embed_body(xc_ref, xl_ref, sc, sqc, sl, sql, g1h, b1h, Wh, bmh,
                    g1l, b1l, Wl, bml, y_ref, ys_ref, ysq_ref):
        mh = sc[...] * ninv
        vh = sqc[...] * ninv - mh * mh
        ah = g1h[...] * lax.rsqrt(vh + EPS_)
        chn = b1h[...] - mh * ah
        yh = jnp.maximum(
            jnp.dot(xc_ref[...] * ah + chn, Wh[...],
                    preferred_element_type=f32) + bmh[...], 0.0)
        ml = sl[...] * ninv
        vl = sql[...] * ninv - ml * ml
        al = g1l[...] * lax.rsqrt(vl + EPS_)
        cln = b1l[...] - ml * al
        yl = jnp.maximum(
            jnp.dot(xl_ref[...] * al + cln, Wl[...],
                    preferred_element_type=f32) + bml[...], 0.0)
        y = jnp.concatenate([yh, yl], axis=1)
        y_ref[...] = y

        @pl.when(pl.program_id(0) == 0)
        def _():
            ys_ref[...] = jnp.zeros_like(ys_ref)
            ysq_ref[...] = jnp.zeros_like(ysq_ref)

        ys_ref[...] += jnp.sum(y, axis=0, keepdims=True)
        ysq_ref[...] += jnp.sum(y * y, axis=0, keepdims=True)

    vec = lambda K: pl.BlockSpec((1, K), lambda i: (0, 0))
    full = lambda a, b: pl.BlockSpec((a, b), lambda i: (0, 0))
    y_, ys_, ysq_ = pl.pallas_call(
        _embed_body,
        grid=(NB,),
        in_specs=[
            pl.BlockSpec((RB, HD), lambda i: (i, 0)),
            pl.BlockSpec((RB, LD), lambda i: (i, 0)),
            vec(HD), vec(HD), vec(LD), vec(LD),
            vec(HD), vec(HD), full(HD, EMB), vec(EMB),
            vec(LD), vec(LD), full(LD, EMB), vec(EMB),
        ],
        out_specs=[
            pl.BlockSpec((RB, 2 * EMB), lambda i: (i, 0)),
            vec(2 * EMB), vec(2 * EMB),
        ],
        out_shape=[
            jax.ShapeDtypeStruct((N, 2 * EMB), f32),
            jax.ShapeDtypeStruct((1, 2 * EMB), f32),
            jax.ShapeDtypeStruct((1, 2 * EMB), f32),
        ],
    )(xc, xl, sc_, sqc_, sl_, sql_,
      r(bn_high_g), r(bn_high_b), mlp_high_W, r(mlp_high_b),
      r(bn_low_g), r(bn_low_b), mlp_low_W, r(mlp_low_b))

    g2 = r(jnp.concatenate([mlp_high_bn_g, mlp_low_bn_g]))
    b2 = r(jnp.concatenate([mlp_high_bn_b, mlp_low_bn_b]))

    def _hprime_body(y_ref, ys, ysq, g2r, b2r, W, p0r, p1r, hp_ref):
        my = ys[...] * ninv
        vy = ysq[...] * ninv - my * my
        a2 = g2r[...] * lax.rsqrt(vy + EPS_)
        c2 = b2r[...] - my * a2
        t = y_ref[...] * a2 + c2
        h = jnp.dot(t, W[...], preferred_element_type=f32)
        deg = 1.0 + p0r[...] + p1r[...]
        hp_ref[...] = lax.rsqrt(deg) * h

    hp = pl.pallas_call(
        _hprime_body,
        grid=(NB,),
        in_specs=[
            pl.BlockSpec((RB, 2 * EMB), lambda i: (i, 0)),
            vec(2 * EMB), vec(2 * EMB), vec(2 * EMB), vec(2 * EMB),
            full(2 * EMB, HID),
            pl.BlockSpec((RB, 1), lambda i: (i, 0)),
            pl.BlockSpec((RB, 1), lambda i: (i, 0)),
        ],
        out_specs=pl.BlockSpec((RB, HID), lambda i: (i, 0)),
        out_shape=jax.ShapeDtypeStruct((N, HID), f32),
    )(y_, ys_, ysq_, g2, b2, conv1_W, p0, p1)

    def _msg_body(hp_hbm, src_hbm, dst_hbm, zeros_hbm, out_hbm,
                  srcv, dstv, rows, acc, gsem0, gsem1):
        c = lax.axis_index("c")
        s = lax.axis_index("s")
        w = c * NS + s
        pltpu.sync_copy(zeros_hbm, acc.at[pl.ds(s * RPT, RPT)])
        plsc.subcore_barrier()

        for p in range(NPH):
            base = w * NF + p * NF2
            pltpu.sync_copy(src_hbm.at[pl.ds(base, NF2)], srcv)
            pltpu.sync_copy(dst_hbm.at[pl.ds(base, NF2)], dstv)
            pltpu.async_copy(hp_hbm.at[srcv.at[0]], rows.at[0], gsem0)

            def step(j2, carry):
                j = 2 * j2
                pltpu.async_copy(hp_hbm.at[srcv.at[j + 1]], rows.at[1], gsem1)
                pltpu.make_async_copy(hp_hbm.at[srcv.at[j]], rows.at[0],
                                      gsem0).wait()
                pltpu.sync_copy(rows.at[0], acc.at[dstv.at[j]], add=True)

                @pl.when(j2 + 1 < NF2 // 2)
                def _():
                    pltpu.async_copy(hp_hbm.at[srcv.at[j + 2]], rows.at[0],
                                     gsem0)

                pltpu.make_async_copy(hp_hbm.at[srcv.at[j + 1]], rows.at[1],
                                      gsem1).wait()
                pltpu.sync_copy(rows.at[1], acc.at[dstv.at[j + 1]], add=True)
                return carry

            lax.fori_loop(0, NF2 // 2, step, 0)

        plsc.subcore_barrier()
        pltpu.sync_copy(acc.at[pl.ds(s * RPT, RPT)],
                        out_hbm.at[c].at[pl.ds(s * RPT, RPT)])

    msg_fn = pl.kernel(
        _msg_body,
        out_type=jax.ShapeDtypeStruct((NC, N_PAD, HID), f32),
        mesh=mesh,
        scratch_types=[
            pltpu.VMEM((NF2, CH), jnp.int32),
            pltpu.VMEM((NF2, CH), jnp.int32),
            pltpu.VMEM((2, CH, HID), f32),
            pltpu.VMEM_SHARED((N_PAD, HID), f32),
            pltpu.SemaphoreType.DMA,
            pltpu.SemaphoreType.DMA,
        ],
    )
    P = msg_fn(hp, src2, dst2, jnp.zeros((RPT, HID), f32))

    def _final_body(P0r, P1r, hpr, p0r, p1r, cb, cW, cbb, o_ref):
        deg = 1.0 + p0r[...] + p1r[...]
        dis = lax.rsqrt(deg)
        x1 = dis * (P0r[...][0] + P1r[...][0] + hpr[...]) + cb[...]
        t = jnp.tanh(x1)
        z = jnp.dot(t, cW[...], preferred_element_type=f32) + cbb[...]
        m = jnp.max(z, axis=1, keepdims=True)
        e = z - m
        o_ref[...] = e - jnp.log(jnp.sum(jnp.exp(e), axis=1, keepdims=True))

    out = pl.pallas_call(
        _final_body,
        grid=(NB,),
        in_specs=[
            pl.BlockSpec((1, RB, HID), lambda i: (0, i, 0)),
            pl.BlockSpec((1, RB, HID), lambda i: (1, i, 0)),
            pl.BlockSpec((RB, HID), lambda i: (i, 0)),
            pl.BlockSpec((RB, 1), lambda i: (i, 0)),
            pl.BlockSpec((RB, 1), lambda i: (i, 0)),
            vec(HID), full(HID, OUT), vec(OUT),
        ],
        out_specs=pl.BlockSpec((RB, OUT), lambda i: (i, 0)),
        out_shape=jax.ShapeDtypeStruct((N, OUT), f32),
    )(P, P, hp, p0, p1, r(conv1_b), cls_W, r(cls_b))

    return out
